# baseline (speedup 1.0000x reference)
import functools

import jax
import jax.numpy as jnp
from jax import lax
from jax.experimental import pallas as pl
from jax.experimental.pallas import tpu as pltpu
from jax.experimental.pallas import tpu_sc as plsc

_N = 10000
_D = 128
_E = 320000
_NTILES = 16
_NPAD = 10240
_RANGE = 5120
_NPH = 2
_ACC = _RANGE + 128
_DUMP = _RANGE
_ACC_PER_TILE = _ACC // _NTILES
_C = 80
_NCH = 250
_NBUF = 2
_ROWBLK = 1024
_NBLK = (2 * _NPAD) // _ROWBLK


def _sc_mesh():
    return plsc.VectorSubcoreMesh(core_axis_name="c", subcore_axis_name="s")


@functools.partial(
    pl.kernel,
    out_type=(
        jax.ShapeDtypeStruct((2 * _NPH, _ACC, _D), jnp.float32),
        jax.ShapeDtypeStruct((2, _NPAD), jnp.float32),
    ),
    mesh=_sc_mesh(),
    scratch_types=[
        pltpu.VMEM((_NCH, _C), jnp.int32),
        pltpu.VMEM((_NCH, _C), jnp.int32),
        pltpu.VMEM((8, _C), jnp.int32),
        pltpu.VMEM((_C, _D), jnp.float32),
        pltpu.VMEM((_C, _D), jnp.float32),
        pltpu.VMEM((_C,), jnp.float32),
        pltpu.VMEM_SHARED((_ACC, _D), jnp.float32),
        pltpu.VMEM_SHARED((_NPAD,), jnp.float32),
        pltpu.SemaphoreType.DMA,
        pltpu.SemaphoreType.DMA,
        pltpu.SemaphoreType.DMA,
        pltpu.SemaphoreType.DMA,
        pltpu.SemaphoreType.DMA,
    ],
)
def _sc_segment_sum(table, srcs, dsts, zrows, zdeg, ones,
                    agg_out, deg_out,
                    src_v, dst_v, adj_v, rows0, rows1, ones_v,
                    acc_sh, deg_sh,
                    sg0, sg1, ss0, ss1, sem_d):
    rows = (rows0, rows1)
    sem_g = (sg0, sg1)
    sem_s = (ss0, ss1)
    c = lax.axis_index("c")
    s = lax.axis_index("s")
    tid = c * _NTILES + s

    pltpu.sync_copy(srcs.at[tid], src_v)
    pltpu.sync_copy(dsts.at[tid], dst_v)
    pltpu.sync_copy(ones, ones_v)

    def gather_start(m, b):
        pltpu.async_copy(table.at[src_v.at[m]], rows[b], sem_g[b])

    def gather_wait(m, b):
        pltpu.make_async_copy(table.at[src_v.at[m]], rows[b], sem_g[b]).wait()

    def adj_row(m):
        return adj_v.at[m % 8]

    def scatter_start(m, b):
        pltpu.async_copy(rows[b], acc_sh.at[adj_row(m)], sem_s[b], add=True)

    def scatter_wait(m, b):
        pltpu.make_async_copy(rows[b], acc_sh.at[adj_row(m)],
                              sem_s[b]).wait()

    def deg_start(m):
        pltpu.async_copy(ones_v, deg_sh.at[dst_v.at[m]], sem_d, add=True)

    def deg_wait(m):
        pltpu.make_async_copy(ones_v, deg_sh.at[dst_v.at[m]], sem_d).wait()

    for p in range(_NPH):
        pltpu.sync_copy(zrows,
                        acc_sh.at[pl.ds(s * _ACC_PER_TILE, _ACC_PER_TILE)])

        if p == 0:
            @pl.when(s == 0)
            def _():
                pltpu.sync_copy(zdeg, deg_sh)

        plsc.subcore_barrier()

        gather_start(0, 0)

        def body(k, carry):
            for t in range(_NBUF):
                m = _NBUF * k + t
                bt = t
                b_nxt = (t + 1) % _NBUF

                if t == 0:
                    @pl.when(k >= 1)
                    def _():
                        scatter_wait(m - 1, b_nxt)
                else:
                    scatter_wait(m - 1, b_nxt)
                if p == 0:
                    @pl.when(k >= 2)
                    def _():
                        deg_wait(m - 4)
                if t == 0:
                    gather_start(m + 1, b_nxt)
                else:
                    @pl.when(k < (_NCH // _NBUF) - 1)
                    def _():
                        gather_start(m + 1, b_nxt)

                gather_wait(m, bt)
                dst_row = dst_v.at[m]
                out_row = adj_row(m)
                for j in range(_C // 16):
                    v = dst_row[pl.ds(j * 16, 16)] - p * _RANGE
                    v = jnp.where(v < 0, _DUMP, v)
                    v = jnp.minimum(v, _DUMP)
                    out_row[pl.ds(j * 16, 16)] = v
                if p == 0:
                    deg_start(m)
                scatter_start(m, bt)
            return carry

        lax.fori_loop(0, _NCH // _NBUF, body, 0)

        scatter_wait(_NCH - 1, (_NCH - 1) % _NBUF)
        if p == 0:
            for m in range(_NCH - 4, _NCH):
                deg_wait(m)

        plsc.subcore_barrier()
        q = c * _NPH + p
        pltpu.sync_copy(
            acc_sh.at[pl.ds(s * _ACC_PER_TILE, _ACC_PER_TILE)],
            agg_out.at[q, pl.ds(s * _ACC_PER_TILE, _ACC_PER_TILE)])

        if p == 0:
            @pl.when(s == 0)
            def _():
                pltpu.sync_copy(deg_sh, deg_out.at[c])
        plsc.subcore_barrier()


def _mm_body(x_ref, w_ref, o_ref):
    o_ref[...] = jnp.dot(x_ref[...], w_ref[...],
                         preferred_element_type=jnp.float32)


def _tc_matmul(x, w):
    return pl.pallas_call(
        _mm_body,
        grid=(_NBLK,),
        in_specs=[
            pl.BlockSpec((_ROWBLK, _D), lambda i: (i, 0)),
            pl.BlockSpec((_D, _D), lambda i: (0, 0)),
        ],
        out_specs=pl.BlockSpec((_ROWBLK, _D), lambda i: (i, 0)),
        out_shape=jax.ShapeDtypeStruct((2 * _NPAD, _D), jnp.float32),
    )(x, w)


def _layer_body(relu, next_w, x_ref, agg_ref, deg_ref, ws_ref, b_ref,
                *rest):
    if next_w:
        wn_ref, h_ref, t_ref = rest
    else:
        (h_ref,) = rest
    rdeg = 1.0 / jnp.maximum(deg_ref[...], 1.0)
    h = (jnp.dot(x_ref[...], ws_ref[...], preferred_element_type=jnp.float32)
         + agg_ref[...] * rdeg + b_ref[...])
    if relu:
        h = jnp.maximum(h, 0.0)
    h_ref[...] = h
    if next_w:
        t_ref[...] = jnp.dot(h, wn_ref[...],
                             preferred_element_type=jnp.float32)


def _tc_layer(x, agg, deg, w_self, b, relu, w_next=None):
    full = jax.ShapeDtypeStruct((2 * _NPAD, _D), jnp.float32)
    full_spec = pl.BlockSpec((_ROWBLK, _D), lambda i: (i, 0))
    in_specs = [
        full_spec,
        full_spec,
        pl.BlockSpec((_ROWBLK, 1), lambda i: (i, 0)),
        pl.BlockSpec((_D, _D), lambda i: (0, 0)),
        pl.BlockSpec((1, _D), lambda i: (0, 0)),
    ]
    out_shape = [full]
    out_specs = [full_spec]
    args = [x, agg, deg.reshape(2 * _NPAD, 1), w_self, b.reshape(1, _D)]
    if w_next is not None:
        in_specs.append(pl.BlockSpec((_D, _D), lambda i: (0, 0)))
        out_shape.append(full)
        out_specs.append(full_spec)
        args.append(w_next)
    outs = pl.pallas_call(
        functools.partial(_layer_body, relu, w_next is not None),
        grid=(_NBLK,),
        in_specs=in_specs,
        out_specs=out_specs,
        out_shape=out_shape,
    )(*args)
    return outs if w_next is not None else outs[0]


def _agg_layer(table, srcs, dsts, zrows, zdeg, ones):
    agg6, deg2 = _sc_segment_sum(table, srcs, dsts, zrows, zdeg, ones)
    last = _NPAD - (_NPH - 1) * _RANGE
    pieces_a = []
    for c in range(2):
        for p in range(_NPH):
            n = _RANGE if p < _NPH - 1 else last
            pieces_a.append(agg6[c * _NPH + p, :n])
    return jnp.concatenate(pieces_a), deg2.reshape(2 * _NPAD)


def _tiles(idx):
    return idx.reshape(_NTILES, _NCH, _C)


def kernel(x_l1, x_l0, edge_index0, edge_index1, edge_index2,
           W_self0, W_neigh0, b0, W_self1, W_neigh1, b1):
    zrows = jnp.zeros((_ACC_PER_TILE, _D), jnp.float32)
    zdeg = jnp.zeros((_NPAD,), jnp.float32)
    ones = jnp.ones((_C,), jnp.float32)

    src0, dst0 = edge_index0[0], edge_index0[1]
    src1, dst1 = edge_index1[0], edge_index1[1]
    src2, dst2 = edge_index2[0], edge_index2[1]
    srcsA = jnp.concatenate([_tiles(src0), _tiles(src1 + _NPAD)])
    dstsA = jnp.concatenate([_tiles(dst0), _tiles(dst1)])
    srcsB = jnp.concatenate([_tiles(src1), _tiles(src2 + _NPAD)])
    dstsB = jnp.concatenate([_tiles(dst1), _tiles(dst2)])

    pad = jnp.zeros((_NPAD - _N, _D), jnp.float32)
    xs = jnp.concatenate([x_l1, pad, x_l0, pad])

    table0 = _tc_matmul(xs, W_neigh0)
    agg0, deg0 = _agg_layer(table0, srcsA, dstsA, zrows, zdeg, ones)
    h, table1 = _tc_layer(xs, agg0, deg0, W_self0, b0, relu=True,
                          w_next=W_neigh1)

    agg1, deg1 = _agg_layer(table1, srcsB, dstsB, zrows, zdeg, ones)
    out = _tc_layer(h, agg1, deg1, W_self1, b1, relu=False)

    h_neib = out[:_N]
    h_seed = out[_NPAD:_NPAD + _N]
    return (h_seed, h_neib)


# speedup vs baseline: 1.0332x; 1.0332x over previous
import functools

import jax
import jax.numpy as jnp
from jax import lax
from jax.experimental import pallas as pl
from jax.experimental.pallas import tpu as pltpu
from jax.experimental.pallas import tpu_sc as plsc

_N = 10000
_D = 128
_E = 320000
_NTILES = 16
_NPAD = 10240
_RANGE = 5120
_NPH = 2
_ACC = _RANGE + 128
_DUMP = _RANGE
_ACC_PER_TILE = _ACC // _NTILES
_C = 80
_NCH = 250
_NBUF = 2
_ROWBLK = 1024
_NBLK = (2 * _NPAD) // _ROWBLK


def _sc_mesh():
    return plsc.VectorSubcoreMesh(core_axis_name="c", subcore_axis_name="s")


@functools.partial(
    pl.kernel,
    out_type=(
        jax.ShapeDtypeStruct((2 * _NPH, _ACC, _D), jnp.float32),
        jax.ShapeDtypeStruct((2, _NPAD), jnp.float32),
    ),
    mesh=_sc_mesh(),
    scratch_types=[
        pltpu.VMEM((_NCH, _C), jnp.int32),
        pltpu.VMEM((_NCH, _C), jnp.int32),
        pltpu.VMEM((8, _C), jnp.int32),
        pltpu.VMEM((_C, _D), jnp.float32),
        pltpu.VMEM((_C, _D), jnp.float32),
        pltpu.VMEM((_C,), jnp.float32),
        pltpu.VMEM_SHARED((_ACC, _D), jnp.float32),
        pltpu.VMEM_SHARED((_NPAD,), jnp.float32),
        pltpu.SemaphoreType.DMA,
        pltpu.SemaphoreType.DMA,
        pltpu.SemaphoreType.DMA,
        pltpu.SemaphoreType.DMA,
        pltpu.SemaphoreType.DMA,
    ],
)
def _sc_segment_sum(table, srcs, dsts, zrows, zdeg, ones,
                    agg_out, deg_out,
                    src_v, dst_v, adj_v, rows0, rows1, ones_v,
                    acc_sh, deg_sh,
                    sg0, sg1, ss0, ss1, sem_d):
    rows = (rows0, rows1)
    sem_g = (sg0, sg1)
    sem_s = (ss0, ss1)
    c = lax.axis_index("c")
    s = lax.axis_index("s")
    tid = c * _NTILES + s

    pltpu.sync_copy(srcs.at[tid], src_v)
    pltpu.sync_copy(dsts.at[tid], dst_v)
    pltpu.sync_copy(ones, ones_v)

    def gather_start(m, b):
        pltpu.async_copy(table.at[src_v.at[m]], rows[b], sem_g[b])

    def gather_wait(m, b):
        pltpu.make_async_copy(table.at[src_v.at[m]], rows[b], sem_g[b]).wait()

    def adj_row(m):
        return adj_v.at[m % 8]

    def scatter_start(m, b):
        pltpu.async_copy(rows[b], acc_sh.at[adj_row(m)], sem_s[b], add=True)

    def scatter_wait(m, b):
        pltpu.make_async_copy(rows[b], acc_sh.at[adj_row(m)],
                              sem_s[b]).wait()

    def deg_start(m):
        pltpu.async_copy(ones_v, deg_sh.at[dst_v.at[m]], sem_d, add=True)

    def deg_wait(m):
        pltpu.make_async_copy(ones_v, deg_sh.at[dst_v.at[m]], sem_d).wait()

    for p in range(_NPH):
        pltpu.sync_copy(zrows,
                        acc_sh.at[pl.ds(s * _ACC_PER_TILE, _ACC_PER_TILE)])

        if p == 0:
            @pl.when(s == 0)
            def _():
                pltpu.sync_copy(zdeg, deg_sh)

        plsc.subcore_barrier()

        gather_start(0, 0)

        def body(k, carry):
            for t in range(_NBUF):
                m = _NBUF * k + t
                bt = t
                b_nxt = (t + 1) % _NBUF

                if t == 0:
                    @pl.when(k >= 1)
                    def _():
                        scatter_wait(m - 1, b_nxt)
                else:
                    scatter_wait(m - 1, b_nxt)
                if p == 0:
                    @pl.when(k >= 2)
                    def _():
                        deg_wait(m - 4)
                if t == 0:
                    gather_start(m + 1, b_nxt)
                else:
                    @pl.when(k < (_NCH // _NBUF) - 1)
                    def _():
                        gather_start(m + 1, b_nxt)

                gather_wait(m, bt)
                dst_row = dst_v.at[m]
                out_row = adj_row(m)
                for j in range(_C // 16):
                    v = dst_row[pl.ds(j * 16, 16)] - p * _RANGE
                    v = jnp.where(v < 0, _DUMP, v)
                    v = jnp.minimum(v, _DUMP)
                    out_row[pl.ds(j * 16, 16)] = v
                if p == 0:
                    deg_start(m)
                scatter_start(m, bt)
            return carry

        lax.fori_loop(0, _NCH // _NBUF, body, 0)

        scatter_wait(_NCH - 1, (_NCH - 1) % _NBUF)
        if p == 0:
            for m in range(_NCH - 4, _NCH):
                deg_wait(m)

        plsc.subcore_barrier()
        q = c * _NPH + p
        pltpu.sync_copy(
            acc_sh.at[pl.ds(s * _ACC_PER_TILE, _ACC_PER_TILE)],
            agg_out.at[q, pl.ds(s * _ACC_PER_TILE, _ACC_PER_TILE)])

        if p == 0:
            @pl.when(s == 0)
            def _():
                pltpu.sync_copy(deg_sh, deg_out.at[c])
        plsc.subcore_barrier()


def _mm_body(x_ref, w_ref, o_ref):
    o_ref[...] = jnp.dot(x_ref[...], w_ref[...],
                         preferred_element_type=jnp.float32)


def _tc_matmul(x, w):
    return pl.pallas_call(
        _mm_body,
        grid=(_NBLK,),
        in_specs=[
            pl.BlockSpec((_ROWBLK, _D), lambda i: (i, 0)),
            pl.BlockSpec((_D, _D), lambda i: (0, 0)),
        ],
        out_specs=pl.BlockSpec((_ROWBLK, _D), lambda i: (i, 0)),
        out_shape=jax.ShapeDtypeStruct((2 * _NPAD, _D), jnp.float32),
    )(x, w)


def _layer_body(relu, next_w, x_ref, agg_ref, deg_ref, ws_ref, b_ref,
                *rest):
    if next_w:
        wn_ref, h_ref, t_ref = rest
    else:
        (h_ref,) = rest
    rdeg = 1.0 / jnp.maximum(deg_ref[...], 1.0)
    h = (jnp.dot(x_ref[...], ws_ref[...], preferred_element_type=jnp.float32)
         + agg_ref[0] * rdeg + b_ref[...])
    if relu:
        h = jnp.maximum(h, 0.0)
    h_ref[...] = h
    if next_w:
        t_ref[...] = jnp.dot(h, wn_ref[...],
                             preferred_element_type=jnp.float32)


def _tc_layer(x, agg, deg, w_self, b, relu, w_next=None):
    full = jax.ShapeDtypeStruct((2 * _NPAD, _D), jnp.float32)
    full_spec = pl.BlockSpec((_ROWBLK, _D), lambda i: (i, 0))
    blk_per_phase = _RANGE // _ROWBLK
    blk_per_pass = _NPAD // _ROWBLK
    in_specs = [
        full_spec,
        pl.BlockSpec((1, _ROWBLK, _D),
                     lambda i: (_NPH * (i // blk_per_pass)
                                + (i % blk_per_pass) // blk_per_phase,
                                (i % blk_per_pass) % blk_per_phase, 0)),
        pl.BlockSpec((_ROWBLK, 1), lambda i: (i, 0)),
        pl.BlockSpec((_D, _D), lambda i: (0, 0)),
        pl.BlockSpec((1, _D), lambda i: (0, 0)),
    ]
    out_shape = [full]
    out_specs = [full_spec]
    args = [x, agg, deg.reshape(2 * _NPAD, 1), w_self, b.reshape(1, _D)]
    if w_next is not None:
        in_specs.append(pl.BlockSpec((_D, _D), lambda i: (0, 0)))
        out_shape.append(full)
        out_specs.append(full_spec)
        args.append(w_next)
    outs = pl.pallas_call(
        functools.partial(_layer_body, relu, w_next is not None),
        grid=(_NBLK,),
        in_specs=in_specs,
        out_specs=out_specs,
        out_shape=out_shape,
    )(*args)
    return outs if w_next is not None else outs[0]


def _agg_layer(table, srcs, dsts, zrows, zdeg, ones):
    agg4, deg2 = _sc_segment_sum(table, srcs, dsts, zrows, zdeg, ones)
    return agg4, deg2.reshape(2 * _NPAD)


def _tiles(idx):
    return idx.reshape(_NTILES, _NCH, _C)


def kernel(x_l1, x_l0, edge_index0, edge_index1, edge_index2,
           W_self0, W_neigh0, b0, W_self1, W_neigh1, b1):
    zrows = jnp.zeros((_ACC_PER_TILE, _D), jnp.float32)
    zdeg = jnp.zeros((_NPAD,), jnp.float32)
    ones = jnp.ones((_C,), jnp.float32)

    src0, dst0 = edge_index0[0], edge_index0[1]
    src1, dst1 = edge_index1[0], edge_index1[1]
    src2, dst2 = edge_index2[0], edge_index2[1]
    srcsA = jnp.concatenate([_tiles(src0), _tiles(src1 + _NPAD)])
    dstsA = jnp.concatenate([_tiles(dst0), _tiles(dst1)])
    srcsB = jnp.concatenate([_tiles(src1), _tiles(src2 + _NPAD)])
    dstsB = jnp.concatenate([_tiles(dst1), _tiles(dst2)])

    pad = jnp.zeros((_NPAD - _N, _D), jnp.float32)
    xs = jnp.concatenate([x_l1, pad, x_l0, pad])

    table0 = _tc_matmul(xs, W_neigh0)
    agg0, deg0 = _agg_layer(table0, srcsA, dstsA, zrows, zdeg, ones)
    h, table1 = _tc_layer(xs, agg0, deg0, W_self0, b0, relu=True,
                          w_next=W_neigh1)

    agg1, deg1 = _agg_layer(table1, srcsB, dstsB, zrows, zdeg, ones)
    out = _tc_layer(h, agg1, deg1, W_self1, b1, relu=False)

    h_neib = out[:_N]
    h_seed = out[_NPAD:_NPAD + _N]
    return (h_seed, h_neib)
